# Initial kernel scaffold; baseline (speedup 1.0000x reference)
import jax
import jax.numpy as jnp
from jax.experimental import pallas as pl


def kernel(dense_input, sparse_input, w, Wd, bd):
    raise NotImplementedError("write your pallas kernel here")


# speedup vs baseline: 1.3909x; 1.3909x over previous
import functools

import jax
import jax.numpy as jnp
from jax import lax
from jax.experimental import pallas as pl
from jax.experimental.pallas import tpu as pltpu
from jax.experimental.pallas import tpu_sc as plsc

B = 16384
N_DENSE = 13
N_SPARSE = 26
NUM_WORKERS = 32
ROWS_PER_W = B // NUM_WORKERS
LANES = 16
SLICES = ROWS_PER_W // LANES

IDX_PER_W = ROWS_PER_W * N_SPARSE
DEN_PER_W = ROWS_PER_W * N_DENSE


def _body(sparse_hbm, dense_hbm, w_hbm, wdb_hbm, out_hbm,
          idx_v, vals_v, den_v, out_v, wdb_v, sem):
    wid = lax.axis_index("s") * 2 + lax.axis_index("c")

    pltpu.sync_copy(sparse_hbm.at[wid], idx_v)
    pltpu.sync_copy(dense_hbm.at[wid], den_v)
    pltpu.sync_copy(wdb_hbm, wdb_v)

    pltpu.async_copy(w_hbm.at[idx_v], vals_v, sem).wait()

    bdv = wdb_v[pl.ds(N_DENSE * LANES, LANES)]

    def slice_body(i, carry):
        r0 = i * LANES
        s = bdv
        for f in range(N_SPARSE):
            s = s + vals_v[pl.ds(f * ROWS_PER_W + r0, LANES)]
        for j in range(N_DENSE):
            s = s + (wdb_v[pl.ds(j * LANES, LANES)]
                     * den_v[pl.ds(j * ROWS_PER_W + r0, LANES)])
        out_v[pl.ds(r0, LANES)] = s
        return carry

    lax.fori_loop(0, SLICES, slice_body, 0)

    pltpu.sync_copy(out_v, out_hbm.at[pl.ds(wid * ROWS_PER_W, ROWS_PER_W)])


@functools.partial(
    pl.kernel,
    out_type=jax.ShapeDtypeStruct((B,), jnp.float32),
    mesh=plsc.VectorSubcoreMesh(core_axis_name="c", subcore_axis_name="s"),
    scratch_types=[
        pltpu.VMEM((IDX_PER_W,), jnp.int32),
        pltpu.VMEM((IDX_PER_W,), jnp.float32),
        pltpu.VMEM((DEN_PER_W,), jnp.float32),
        pltpu.VMEM((ROWS_PER_W,), jnp.float32),
        pltpu.VMEM(((N_DENSE + 1) * LANES,), jnp.float32),
        pltpu.SemaphoreType.DMA,
    ],
)
def _sc_kernel(sparse_hbm, dense_hbm, w_hbm, wdb_hbm, out_hbm,
               idx_v, vals_v, den_v, out_v, wdb_v, sem):
    _body(sparse_hbm, dense_hbm, w_hbm, wdb_hbm, out_hbm,
          idx_v, vals_v, den_v, out_v, wdb_v, sem)


def kernel(dense_input, sparse_input, w, Wd, bd):
    sparse_t = jnp.transpose(
        sparse_input.astype(jnp.int32).reshape(NUM_WORKERS, ROWS_PER_W, N_SPARSE),
        (0, 2, 1)).reshape(NUM_WORKERS, IDX_PER_W)
    dense_t = jnp.transpose(
        dense_input.reshape(NUM_WORKERS, ROWS_PER_W, N_DENSE),
        (0, 2, 1)).reshape(NUM_WORKERS, DEN_PER_W)
    w_flat = w.reshape(-1)
    wdb = jnp.concatenate(
        [jnp.repeat(Wd.reshape(-1), LANES), jnp.repeat(bd.reshape(-1), LANES)])
    out = _sc_kernel(sparse_t, dense_t, w_flat, wdb)
    return out.reshape(B, 1)
